# baseline (speedup 1.0000x reference)
import functools

import jax
import jax.numpy as jnp
from jax.experimental import pallas as pl
from jax.experimental.pallas import tpu as pltpu

N_CODES = 8192
CODE_DIM = 256
COMMITMENT_COST = 0.25

M_BLK = 1024
K_BLK = 2048


C_BLK = 512


def _argmin_body(zsq_ref, wsq_ref, z2b_ref, wb_ref, idx_out,
                 minval_s, minidx_s):
    k = pl.program_id(1)
    nk = pl.num_programs(1)

    zsq = zsq_ref[...]
    z2b = z2b_ref[...]
    big = jnp.int32(jnp.iinfo(jnp.int32).max)
    iota_c = jax.lax.broadcasted_iota(jnp.int32, (M_BLK, C_BLK), 1)

    lm = None
    li = None
    for c in range(K_BLK // C_BLK):
        wb_c = wb_ref[pl.ds(c * C_BLK, C_BLK), :]
        mm_c = jax.lax.dot_general(
            z2b, wb_c,
            dimension_numbers=(((1,), (1,)), ((), ())),
            preferred_element_type=jnp.float32)
        dc = (zsq + wsq_ref[:, pl.ds(c * C_BLK, C_BLK)]) - mm_c
        lm_c = jnp.min(dc, axis=1, keepdims=True)
        li_loc = jnp.min(jnp.where(dc == lm_c, iota_c, big),
                         axis=1, keepdims=True)
        li_c = li_loc + (k * K_BLK + c * C_BLK)
        if c == 0:
            lm, li = lm_c, li_c
        else:
            better = lm_c < lm
            li = jnp.where(better, li_c, li)
            lm = jnp.where(better, lm_c, lm)

    @pl.when(k == 0)
    def _():
        minval_s[...] = lm.astype(jnp.bfloat16).astype(jnp.float32)
        minidx_s[...] = li

    @pl.when(k > 0)
    def _():
        m = minval_s[...]
        i = minidx_s[...]
        better = (lm < m) | ((lm == m) & (li < i))
        minidx_s[...] = jnp.where(better, li, i)
        minval_s[...] = jnp.where(better, lm, m).astype(
            jnp.bfloat16).astype(jnp.float32)

    @pl.when(k == nk - 1)
    def _():
        idx_out[...] = minidx_s[...]


def _argmin_call(z2b, wb, zsq, wsq):
    M = z2b.shape[0]
    grid = (M // M_BLK, N_CODES // K_BLK)
    return pl.pallas_call(
        _argmin_body,
        grid=grid,
        in_specs=[
            pl.BlockSpec((M_BLK, 1), lambda m, k: (m, 0)),
            pl.BlockSpec((1, K_BLK), lambda m, k: (0, k)),
            pl.BlockSpec((M_BLK, CODE_DIM), lambda m, k: (m, 0)),
            pl.BlockSpec((K_BLK, CODE_DIM), lambda m, k: (k, 0)),
        ],
        out_specs=pl.BlockSpec((M_BLK, 1), lambda m, k: (m, 0)),
        out_shape=jax.ShapeDtypeStruct((M, 1), jnp.int32),
        scratch_shapes=[
            pltpu.VMEM((M_BLK, 1), jnp.float32),
            pltpu.VMEM((M_BLK, 1), jnp.int32),
        ],
    )(zsq, wsq, z2b, wb)


def kernel(z, W):
    B, N, D = z.shape
    z_flat = z.reshape(-1, D)
    zsq = jnp.sum(z_flat ** 2, axis=1, keepdims=True)
    wsq = jnp.sum(W ** 2, axis=1)[None, :]
    z2b = (2.0 * z_flat).astype(jnp.bfloat16)
    wb = W.astype(jnp.bfloat16)

    idx2 = _argmin_call(z2b, wb, zsq, wsq)
    indices_flat = idx2[:, 0]
    indices = indices_flat.reshape(B, N)

    z_q_flat = jnp.take(W, indices_flat, axis=0)
    z_q = z_q_flat.reshape(B, N, D)
    commitment_loss = jnp.mean((z - jax.lax.stop_gradient(z_q)) ** 2)
    codebook_loss = jnp.mean((z_q - jax.lax.stop_gradient(z)) ** 2)
    vq_loss = commitment_loss * COMMITMENT_COST + codebook_loss
    z_q_st = z + jax.lax.stop_gradient(z_q - z)
    counts = jnp.bincount(indices_flat, length=N_CODES).astype(jnp.float32)
    avg_probs = counts / indices_flat.shape[0]
    perplexity = jnp.exp(-jnp.sum(avg_probs * jnp.log(avg_probs + 1e-10)))
    return (z_q_st, indices, vq_loss, perplexity)


# speedup vs baseline: 1.2748x; 1.2748x over previous
import functools

import jax
import jax.numpy as jnp
from jax.experimental import pallas as pl
from jax.experimental.pallas import tpu as pltpu

N_CODES = 8192
CODE_DIM = 256
COMMITMENT_COST = 0.25

M_BLK = 1024
K_BLK = 2048


C_BLK = 512


def _argmin_body(zsq_ref, wsq_ref, z2b_ref, wb_ref, idx_out,
                 minval_s, minidx_s):
    k = pl.program_id(1)
    nk = pl.num_programs(1)

    zsq = zsq_ref[...]
    z2b = z2b_ref[...]
    big = jnp.int32(jnp.iinfo(jnp.int32).max)
    iota128 = jax.lax.broadcasted_iota(jnp.int32, (M_BLK, 128), 1)

    acc_v = None
    acc_c = None
    for c in range(K_BLK // C_BLK):
        wb_c = wb_ref[pl.ds(c * C_BLK, C_BLK), :]
        mm_c = jax.lax.dot_general(
            z2b, wb_c,
            dimension_numbers=(((1,), (1,)), ((), ())),
            preferred_element_type=jnp.float32)
        dc = (zsq + wsq_ref[:, pl.ds(c * C_BLK, C_BLK)]) - mm_c
        for h in range(C_BLK // 128):
            cand = dc[:, h * 128:(h + 1) * 128]
            vid = c * (C_BLK // 128) + h
            if acc_v is None:
                acc_v = cand
                acc_c = jnp.zeros((M_BLK, 128), jnp.int32)
            else:
                better = cand < acc_v
                acc_v = jnp.where(better, cand, acc_v)
                acc_c = jnp.where(better,
                                  jnp.full((M_BLK, 128), vid, jnp.int32),
                                  acc_c)

    lm = jnp.min(acc_v, axis=1, keepdims=True)
    col = acc_c * 128 + iota128
    li = jnp.min(jnp.where(acc_v == lm, col, big),
                 axis=1, keepdims=True) + k * K_BLK

    @pl.when(k == 0)
    def _():
        minval_s[...] = lm.astype(jnp.bfloat16).astype(jnp.float32)
        minidx_s[...] = li

    @pl.when(k > 0)
    def _():
        m = minval_s[...]
        i = minidx_s[...]
        better = (lm < m) | ((lm == m) & (li < i))
        minidx_s[...] = jnp.where(better, li, i)
        minval_s[...] = jnp.where(better, lm, m).astype(
            jnp.bfloat16).astype(jnp.float32)

    @pl.when(k == nk - 1)
    def _():
        idx_out[...] = minidx_s[...]


def _argmin_call(z2b, wb, zsq, wsq):
    M = z2b.shape[0]
    grid = (M // M_BLK, N_CODES // K_BLK)
    return pl.pallas_call(
        _argmin_body,
        grid=grid,
        in_specs=[
            pl.BlockSpec((M_BLK, 1), lambda m, k: (m, 0)),
            pl.BlockSpec((1, K_BLK), lambda m, k: (0, k)),
            pl.BlockSpec((M_BLK, CODE_DIM), lambda m, k: (m, 0)),
            pl.BlockSpec((K_BLK, CODE_DIM), lambda m, k: (k, 0)),
        ],
        out_specs=pl.BlockSpec((M_BLK, 1), lambda m, k: (m, 0)),
        out_shape=jax.ShapeDtypeStruct((M, 1), jnp.int32),
        scratch_shapes=[
            pltpu.VMEM((M_BLK, 1), jnp.float32),
            pltpu.VMEM((M_BLK, 1), jnp.int32),
        ],
    )(zsq, wsq, z2b, wb)


def kernel(z, W):
    B, N, D = z.shape
    z_flat = z.reshape(-1, D)
    zsq = jnp.sum(z_flat ** 2, axis=1, keepdims=True)
    wsq = jnp.sum(W ** 2, axis=1)[None, :]
    z2b = (2.0 * z_flat).astype(jnp.bfloat16)
    wb = W.astype(jnp.bfloat16)

    idx2 = _argmin_call(z2b, wb, zsq, wsq)
    indices_flat = idx2[:, 0]
    indices = indices_flat.reshape(B, N)

    z_q_flat = jnp.take(W, indices_flat, axis=0)
    z_q = z_q_flat.reshape(B, N, D)
    commitment_loss = jnp.mean((z - jax.lax.stop_gradient(z_q)) ** 2)
    codebook_loss = jnp.mean((z_q - jax.lax.stop_gradient(z)) ** 2)
    vq_loss = commitment_loss * COMMITMENT_COST + codebook_loss
    z_q_st = z + jax.lax.stop_gradient(z_q - z)
    counts = jnp.bincount(indices_flat, length=N_CODES).astype(jnp.float32)
    avg_probs = counts / indices_flat.shape[0]
    perplexity = jnp.exp(-jnp.sum(avg_probs * jnp.log(avg_probs + 1e-10)))
    return (z_q_st, indices, vq_loss, perplexity)


# speedup vs baseline: 1.3627x; 1.0690x over previous
import functools

import jax
import jax.numpy as jnp
from jax import lax
from jax.experimental import pallas as pl
from jax.experimental.pallas import tpu as pltpu
from jax.experimental.pallas import tpu_sc as plsc

N_CODES = 8192
CODE_DIM = 256
COMMITMENT_COST = 0.25

M_BLK = 1024
K_BLK = 2048


C_BLK = 512


def _argmin_body(zsq_ref, wsq_ref, z2b_ref, wb_ref, idx_out,
                 minval_s, minidx_s):
    k = pl.program_id(1)
    nk = pl.num_programs(1)

    zsq = zsq_ref[...]
    z2b = z2b_ref[...]
    big = jnp.int32(jnp.iinfo(jnp.int32).max)
    iota128 = jax.lax.broadcasted_iota(jnp.int32, (M_BLK, 128), 1)

    acc_v = None
    acc_c = None
    for c in range(K_BLK // C_BLK):
        wb_c = wb_ref[pl.ds(c * C_BLK, C_BLK), :]
        mm_c = jax.lax.dot_general(
            z2b, wb_c,
            dimension_numbers=(((1,), (1,)), ((), ())),
            preferred_element_type=jnp.float32)
        dc = (zsq + wsq_ref[:, pl.ds(c * C_BLK, C_BLK)]) - mm_c
        for h in range(C_BLK // 128):
            cand = dc[:, h * 128:(h + 1) * 128]
            vid = c * (C_BLK // 128) + h
            if acc_v is None:
                acc_v = cand
                acc_c = jnp.zeros((M_BLK, 128), jnp.int32)
            else:
                better = cand < acc_v
                acc_v = jnp.where(better, cand, acc_v)
                acc_c = jnp.where(better,
                                  jnp.full((M_BLK, 128), vid, jnp.int32),
                                  acc_c)

    lm = jnp.min(acc_v, axis=1, keepdims=True)
    col = acc_c * 128 + iota128
    li = jnp.min(jnp.where(acc_v == lm, col, big),
                 axis=1, keepdims=True) + k * K_BLK

    @pl.when(k == 0)
    def _():
        minval_s[...] = lm.astype(jnp.bfloat16).astype(jnp.float32)
        minidx_s[...] = li

    @pl.when(k > 0)
    def _():
        m = minval_s[...]
        i = minidx_s[...]
        better = (lm < m) | ((lm == m) & (li < i))
        minidx_s[...] = jnp.where(better, li, i)
        minval_s[...] = jnp.where(better, lm, m).astype(
            jnp.bfloat16).astype(jnp.float32)

    @pl.when(k == nk - 1)
    def _():
        idx_out[...] = minidx_s[...]


def _argmin_call(z2b, wb, zsq, wsq):
    M = z2b.shape[0]
    grid = (M // M_BLK, N_CODES // K_BLK)
    return pl.pallas_call(
        _argmin_body,
        grid=grid,
        in_specs=[
            pl.BlockSpec((M_BLK, 1), lambda m, k: (m, 0)),
            pl.BlockSpec((1, K_BLK), lambda m, k: (0, k)),
            pl.BlockSpec((M_BLK, CODE_DIM), lambda m, k: (m, 0)),
            pl.BlockSpec((K_BLK, CODE_DIM), lambda m, k: (k, 0)),
        ],
        out_specs=pl.BlockSpec((M_BLK, 1), lambda m, k: (m, 0)),
        out_shape=jax.ShapeDtypeStruct((M, 1), jnp.int32),
        scratch_shapes=[
            pltpu.VMEM((M_BLK, 1), jnp.float32),
            pltpu.VMEM((M_BLK, 1), jnp.int32),
        ],
    )(zsq, wsq, z2b, wb)


NW = 32
BPW = 8192 // NW


def _sc_gather_counts(W, idx2d, zeros):
    mesh = plsc.VectorSubcoreMesh(core_axis_name="c", subcore_axis_name="s")

    @functools.partial(
        pl.kernel, mesh=mesh,
        out_type=[
            jax.ShapeDtypeStruct((8192, CODE_DIM), jnp.float32),
            jax.ShapeDtypeStruct((2, N_CODES), jnp.float32),
        ],
        scratch_types=[
            pltpu.VMEM((BPW // 128, 128), jnp.int32),
            pltpu.VMEM((BPW, CODE_DIM), jnp.float32),
            pltpu.VMEM_SHARED((N_CODES,), jnp.float32),
            pltpu.VMEM((128,), jnp.float32),
            pltpu.SemaphoreType.DMA,
        ],
    )
    def body(w_hbm, idx2_hbm, zeros_hbm, zq_hbm, cnt_hbm,
             idx_v, rows_v, cnt_sh, ones_v, sem):
        sid = lax.axis_index("s")
        cid = lax.axis_index("c")
        wid = sid * 2 + cid
        nrow = BPW // 128

        @pl.when(sid == 0)
        def _():
            pltpu.sync_copy(zeros_hbm, cnt_sh)

        pltpu.sync_copy(idx2_hbm.at[pl.ds(wid * nrow, nrow), :], idx_v)
        for i in range(8):
            ones_v[pl.ds(i * 16, 16)] = jnp.ones((16,), jnp.float32)
        plsc.subcore_barrier()
        for j in range(nrow):
            pltpu.async_copy(w_hbm.at[idx_v.at[j]],
                             rows_v.at[pl.ds(j * 128, 128), :], sem).wait()
            pltpu.sync_copy(ones_v, cnt_sh.at[idx_v.at[j]], add=True)
        pltpu.sync_copy(rows_v, zq_hbm.at[pl.ds(wid * BPW, BPW), :])
        plsc.subcore_barrier()

        @pl.when(sid == 0)
        def _():
            pltpu.sync_copy(cnt_sh, cnt_hbm.at[cid])

    return body(W, idx2d, zeros)


EPS_LOG = 1e-10


def _epi_body(z_ref, zq_ref, cnt_ref, st_out, loss_out, perp_out, ss_s):
    m = pl.program_id(0)
    nm = pl.num_programs(0)
    z = z_ref[...]
    zq = zq_ref[...]
    st_out[...] = z + (zq - z)
    diff = z - zq
    s = jnp.sum(diff * diff, keepdims=True)[:1, :1]

    @pl.when(m == 0)
    def _():
        ss_s[...] = s
        c = jnp.sum(cnt_ref[...], axis=0, keepdims=True)
        p = c * (1.0 / 8192.0)
        ent = jnp.sum(p * jnp.log(p + EPS_LOG), keepdims=True)[:1, :1]
        perp_out[...] = jnp.exp(-ent)

    @pl.when(m > 0)
    def _():
        ss_s[...] = ss_s[...] + s

    @pl.when(m == nm - 1)
    def _():
        mean = ss_s[...] * (1.0 / (8192.0 * CODE_DIM))
        loss_out[...] = mean * COMMITMENT_COST + mean


def _epilogue_call(z_flat, zq_flat, counts):
    M = z_flat.shape[0]
    grid = (M // M_BLK,)
    return pl.pallas_call(
        _epi_body,
        grid=grid,
        in_specs=[
            pl.BlockSpec((M_BLK, CODE_DIM), lambda m: (m, 0)),
            pl.BlockSpec((M_BLK, CODE_DIM), lambda m: (m, 0)),
            pl.BlockSpec((2, N_CODES), lambda m: (0, 0)),
        ],
        out_specs=[
            pl.BlockSpec((M_BLK, CODE_DIM), lambda m: (m, 0)),
            pl.BlockSpec((1, 1), lambda m: (0, 0)),
            pl.BlockSpec((1, 1), lambda m: (0, 0)),
        ],
        out_shape=[
            jax.ShapeDtypeStruct((M, CODE_DIM), jnp.float32),
            jax.ShapeDtypeStruct((1, 1), jnp.float32),
            jax.ShapeDtypeStruct((1, 1), jnp.float32),
        ],
        scratch_shapes=[pltpu.VMEM((1, 1), jnp.float32)],
    )(z_flat, zq_flat, counts)


def kernel(z, W):
    B, N, D = z.shape
    z_flat = z.reshape(-1, D)
    zsq = jnp.sum(z_flat ** 2, axis=1, keepdims=True)
    wsq = jnp.sum(W ** 2, axis=1)[None, :]
    z2b = (2.0 * z_flat).astype(jnp.bfloat16)
    wb = W.astype(jnp.bfloat16)

    idx2 = _argmin_call(z2b, wb, zsq, wsq)
    indices_flat = idx2[:, 0]
    indices = indices_flat.reshape(B, N)

    idx2d = indices_flat.reshape(8192 // 128, 128)
    zeros = jnp.zeros((N_CODES,), jnp.float32)
    zq_flat, counts = _sc_gather_counts(W, idx2d, zeros)

    zq_st_flat, loss, perp = _epilogue_call(z_flat, zq_flat, counts)
    z_q_st = zq_st_flat.reshape(B, N, D)
    return (z_q_st, indices, loss.reshape(()), perp.reshape(()))


# speedup vs baseline: 1.4263x; 1.0467x over previous
import functools

import jax
import jax.numpy as jnp
from jax import lax
from jax.experimental import pallas as pl
from jax.experimental.pallas import tpu as pltpu
from jax.experimental.pallas import tpu_sc as plsc

N_CODES = 8192
CODE_DIM = 256
COMMITMENT_COST = 0.25

M_BLK = 2048
K_BLK = 2048


C_BLK = 512


def _argmin_body(zsq_ref, wsq_ref, z2b_ref, wb_ref, idx_out,
                 minval_s, minidx_s):
    k = pl.program_id(1)
    nk = pl.num_programs(1)

    zsq = zsq_ref[...]
    z2b = z2b_ref[...]
    big = jnp.int32(jnp.iinfo(jnp.int32).max)
    iota128 = jax.lax.broadcasted_iota(jnp.int32, (M_BLK, 128), 1)

    acc_v = None
    acc_c = None
    for c in range(K_BLK // C_BLK):
        wb_c = wb_ref[pl.ds(c * C_BLK, C_BLK), :]
        mm_c = jax.lax.dot_general(
            z2b, wb_c,
            dimension_numbers=(((1,), (1,)), ((), ())),
            preferred_element_type=jnp.float32)
        dc = (zsq + wsq_ref[:, pl.ds(c * C_BLK, C_BLK)]) - mm_c
        for h in range(C_BLK // 128):
            cand = dc[:, h * 128:(h + 1) * 128]
            vid = c * (C_BLK // 128) + h
            if acc_v is None:
                acc_v = cand
                acc_c = jnp.zeros((M_BLK, 128), jnp.int32)
            else:
                better = cand < acc_v
                acc_v = jnp.where(better, cand, acc_v)
                acc_c = jnp.where(better,
                                  jnp.full((M_BLK, 128), vid, jnp.int32),
                                  acc_c)

    lm = jnp.min(acc_v, axis=1, keepdims=True)
    col = acc_c * 128 + iota128
    li = jnp.min(jnp.where(acc_v == lm, col, big),
                 axis=1, keepdims=True) + k * K_BLK

    @pl.when(k == 0)
    def _():
        minval_s[...] = lm.astype(jnp.bfloat16).astype(jnp.float32)
        minidx_s[...] = li

    @pl.when(k > 0)
    def _():
        m = minval_s[...]
        i = minidx_s[...]
        better = (lm < m) | ((lm == m) & (li < i))
        minidx_s[...] = jnp.where(better, li, i)
        minval_s[...] = jnp.where(better, lm, m).astype(
            jnp.bfloat16).astype(jnp.float32)

    @pl.when(k == nk - 1)
    def _():
        idx_out[...] = minidx_s[...]


def _argmin_call(z2b, wb, zsq, wsq):
    M = z2b.shape[0]
    grid = (M // M_BLK, N_CODES // K_BLK)
    return pl.pallas_call(
        _argmin_body,
        grid=grid,
        in_specs=[
            pl.BlockSpec((M_BLK, 1), lambda m, k: (m, 0)),
            pl.BlockSpec((1, K_BLK), lambda m, k: (0, k)),
            pl.BlockSpec((M_BLK, CODE_DIM), lambda m, k: (m, 0)),
            pl.BlockSpec((K_BLK, CODE_DIM), lambda m, k: (k, 0)),
        ],
        out_specs=pl.BlockSpec((M_BLK, 1), lambda m, k: (m, 0)),
        out_shape=jax.ShapeDtypeStruct((M, 1), jnp.int32),
        scratch_shapes=[
            pltpu.VMEM((M_BLK, 1), jnp.float32),
            pltpu.VMEM((M_BLK, 1), jnp.int32),
        ],
    )(zsq, wsq, z2b, wb)


NW = 32
BPW = 8192 // NW


def _sc_gather_counts(W, idx2d, zeros):
    mesh = plsc.VectorSubcoreMesh(core_axis_name="c", subcore_axis_name="s")

    @functools.partial(
        pl.kernel, mesh=mesh,
        out_type=[
            jax.ShapeDtypeStruct((8192, CODE_DIM), jnp.float32),
            jax.ShapeDtypeStruct((2, N_CODES), jnp.float32),
        ],
        scratch_types=[
            pltpu.VMEM((BPW // 128, 128), jnp.int32),
            pltpu.VMEM((BPW, CODE_DIM), jnp.float32),
            pltpu.VMEM_SHARED((N_CODES,), jnp.float32),
            pltpu.VMEM((128,), jnp.float32),
            pltpu.SemaphoreType.DMA,
        ],
    )
    def body(w_hbm, idx2_hbm, zeros_hbm, zq_hbm, cnt_hbm,
             idx_v, rows_v, cnt_sh, ones_v, sem):
        sid = lax.axis_index("s")
        cid = lax.axis_index("c")
        wid = sid * 2 + cid
        nrow = BPW // 128

        @pl.when(sid == 0)
        def _():
            pltpu.sync_copy(zeros_hbm, cnt_sh)

        pltpu.sync_copy(idx2_hbm.at[pl.ds(wid * nrow, nrow), :], idx_v)
        for i in range(8):
            ones_v[pl.ds(i * 16, 16)] = jnp.ones((16,), jnp.float32)
        plsc.subcore_barrier()
        for j in range(nrow):
            pltpu.async_copy(w_hbm.at[idx_v.at[j]],
                             rows_v.at[pl.ds(j * 128, 128), :], sem).wait()
            pltpu.sync_copy(ones_v, cnt_sh.at[idx_v.at[j]], add=True)
        pltpu.sync_copy(rows_v, zq_hbm.at[pl.ds(wid * BPW, BPW), :])
        plsc.subcore_barrier()

        @pl.when(sid == 0)
        def _():
            pltpu.sync_copy(cnt_sh, cnt_hbm.at[cid])

    return body(W, idx2d, zeros)


EPS_LOG = 1e-10


def _epi_body(z_ref, zq_ref, cnt_ref, st_out, loss_out, perp_out, ss_s):
    m = pl.program_id(0)
    nm = pl.num_programs(0)
    z = z_ref[...]
    zq = zq_ref[...]
    st_out[...] = z + (zq - z)
    diff = z - zq
    s = jnp.sum(diff * diff, keepdims=True)[:1, :1]

    @pl.when(m == 0)
    def _():
        ss_s[...] = s
        c = jnp.sum(cnt_ref[...], axis=0, keepdims=True)
        p = c * (1.0 / 8192.0)
        ent = jnp.sum(p * jnp.log(p + EPS_LOG), keepdims=True)[:1, :1]
        perp_out[...] = jnp.exp(-ent)

    @pl.when(m > 0)
    def _():
        ss_s[...] = ss_s[...] + s

    @pl.when(m == nm - 1)
    def _():
        mean = ss_s[...] * (1.0 / (8192.0 * CODE_DIM))
        loss_out[...] = mean * COMMITMENT_COST + mean


def _epilogue_call(z_flat, zq_flat, counts):
    M = z_flat.shape[0]
    grid = (M // M_BLK,)
    return pl.pallas_call(
        _epi_body,
        grid=grid,
        in_specs=[
            pl.BlockSpec((M_BLK, CODE_DIM), lambda m: (m, 0)),
            pl.BlockSpec((M_BLK, CODE_DIM), lambda m: (m, 0)),
            pl.BlockSpec((2, N_CODES), lambda m: (0, 0)),
        ],
        out_specs=[
            pl.BlockSpec((M_BLK, CODE_DIM), lambda m: (m, 0)),
            pl.BlockSpec((1, 1), lambda m: (0, 0)),
            pl.BlockSpec((1, 1), lambda m: (0, 0)),
        ],
        out_shape=[
            jax.ShapeDtypeStruct((M, CODE_DIM), jnp.float32),
            jax.ShapeDtypeStruct((1, 1), jnp.float32),
            jax.ShapeDtypeStruct((1, 1), jnp.float32),
        ],
        scratch_shapes=[pltpu.VMEM((1, 1), jnp.float32)],
    )(z_flat, zq_flat, counts)


def kernel(z, W):
    B, N, D = z.shape
    z_flat = z.reshape(-1, D)
    zsq = jnp.sum(z_flat ** 2, axis=1, keepdims=True)
    wsq = jnp.sum(W ** 2, axis=1)[None, :]
    z2b = (2.0 * z_flat).astype(jnp.bfloat16)
    wb = W.astype(jnp.bfloat16)

    idx2 = _argmin_call(z2b, wb, zsq, wsq)
    indices_flat = idx2[:, 0]
    indices = indices_flat.reshape(B, N)

    idx2d = indices_flat.reshape(8192 // 128, 128)
    zeros = jnp.zeros((N_CODES,), jnp.float32)
    zq_flat, counts = _sc_gather_counts(W, idx2d, zeros)

    zq_st_flat, loss, perp = _epilogue_call(z_flat, zq_flat, counts)
    z_q_st = zq_st_flat.reshape(B, N, D)
    return (z_q_st, indices, loss.reshape(()), perp.reshape(()))


# speedup vs baseline: 1.4286x; 1.0016x over previous
import functools

import jax
import jax.numpy as jnp
from jax import lax
from jax.experimental import pallas as pl
from jax.experimental.pallas import tpu as pltpu
from jax.experimental.pallas import tpu_sc as plsc

N_CODES = 8192
CODE_DIM = 256
COMMITMENT_COST = 0.25

M_BLK = 4096
K_BLK = 2048


C_BLK = 512


def _argmin_body(zsq_ref, wsq_ref, z2b_ref, wb_ref, idx_out,
                 minval_s, minidx_s):
    k = pl.program_id(1)
    nk = pl.num_programs(1)

    zsq = zsq_ref[...]
    z2b = z2b_ref[...]
    big = jnp.int32(jnp.iinfo(jnp.int32).max)
    iota128 = jax.lax.broadcasted_iota(jnp.int32, (M_BLK, 128), 1)

    acc_v = None
    acc_c = None
    for c in range(K_BLK // C_BLK):
        wb_c = wb_ref[pl.ds(c * C_BLK, C_BLK), :]
        mm_c = jax.lax.dot_general(
            z2b, wb_c,
            dimension_numbers=(((1,), (1,)), ((), ())),
            preferred_element_type=jnp.float32)
        dc = (zsq + wsq_ref[:, pl.ds(c * C_BLK, C_BLK)]) - mm_c
        for h in range(C_BLK // 128):
            cand = dc[:, h * 128:(h + 1) * 128]
            vid = c * (C_BLK // 128) + h
            if acc_v is None:
                acc_v = cand
                acc_c = jnp.zeros((M_BLK, 128), jnp.int32)
            else:
                better = cand < acc_v
                acc_v = jnp.where(better, cand, acc_v)
                acc_c = jnp.where(better,
                                  jnp.full((M_BLK, 128), vid, jnp.int32),
                                  acc_c)

    lm = jnp.min(acc_v, axis=1, keepdims=True)
    col = acc_c * 128 + iota128
    li = jnp.min(jnp.where(acc_v == lm, col, big),
                 axis=1, keepdims=True) + k * K_BLK

    @pl.when(k == 0)
    def _():
        minval_s[...] = lm.astype(jnp.bfloat16).astype(jnp.float32)
        minidx_s[...] = li

    @pl.when(k > 0)
    def _():
        m = minval_s[...]
        i = minidx_s[...]
        better = (lm < m) | ((lm == m) & (li < i))
        minidx_s[...] = jnp.where(better, li, i)
        minval_s[...] = jnp.where(better, lm, m).astype(
            jnp.bfloat16).astype(jnp.float32)

    @pl.when(k == nk - 1)
    def _():
        idx_out[...] = minidx_s[...]


def _argmin_call(z2b, wb, zsq, wsq):
    M = z2b.shape[0]
    grid = (M // M_BLK, N_CODES // K_BLK)
    return pl.pallas_call(
        _argmin_body,
        grid=grid,
        in_specs=[
            pl.BlockSpec((M_BLK, 1), lambda m, k: (m, 0)),
            pl.BlockSpec((1, K_BLK), lambda m, k: (0, k)),
            pl.BlockSpec((M_BLK, CODE_DIM), lambda m, k: (m, 0)),
            pl.BlockSpec((K_BLK, CODE_DIM), lambda m, k: (k, 0)),
        ],
        out_specs=pl.BlockSpec((M_BLK, 1), lambda m, k: (m, 0)),
        out_shape=jax.ShapeDtypeStruct((M, 1), jnp.int32),
        scratch_shapes=[
            pltpu.VMEM((M_BLK, 1), jnp.float32),
            pltpu.VMEM((M_BLK, 1), jnp.int32),
        ],
    )(zsq, wsq, z2b, wb)


NW = 32
BPW = 8192 // NW


def _sc_gather_counts(W, idx2d, zeros):
    mesh = plsc.VectorSubcoreMesh(core_axis_name="c", subcore_axis_name="s")

    @functools.partial(
        pl.kernel, mesh=mesh,
        out_type=[
            jax.ShapeDtypeStruct((8192, CODE_DIM), jnp.float32),
            jax.ShapeDtypeStruct((2, N_CODES), jnp.float32),
        ],
        scratch_types=[
            pltpu.VMEM((BPW // 128, 128), jnp.int32),
            pltpu.VMEM((BPW, CODE_DIM), jnp.float32),
            pltpu.VMEM_SHARED((N_CODES,), jnp.float32),
            pltpu.VMEM((128,), jnp.float32),
            pltpu.SemaphoreType.DMA,
        ],
    )
    def body(w_hbm, idx2_hbm, zeros_hbm, zq_hbm, cnt_hbm,
             idx_v, rows_v, cnt_sh, ones_v, sem):
        sid = lax.axis_index("s")
        cid = lax.axis_index("c")
        wid = sid * 2 + cid
        nrow = BPW // 128

        @pl.when(sid == 0)
        def _():
            pltpu.sync_copy(zeros_hbm, cnt_sh)

        pltpu.sync_copy(idx2_hbm.at[pl.ds(wid * nrow, nrow), :], idx_v)
        for i in range(8):
            ones_v[pl.ds(i * 16, 16)] = jnp.ones((16,), jnp.float32)
        plsc.subcore_barrier()
        for j in range(nrow):
            pltpu.async_copy(w_hbm.at[idx_v.at[j]],
                             rows_v.at[pl.ds(j * 128, 128), :], sem).wait()
            pltpu.sync_copy(ones_v, cnt_sh.at[idx_v.at[j]], add=True)
        pltpu.sync_copy(rows_v, zq_hbm.at[pl.ds(wid * BPW, BPW), :])
        plsc.subcore_barrier()

        @pl.when(sid == 0)
        def _():
            pltpu.sync_copy(cnt_sh, cnt_hbm.at[cid])

    return body(W, idx2d, zeros)


EPS_LOG = 1e-10


def _epi_body(z_ref, zq_ref, cnt_ref, st_out, loss_out, perp_out, ss_s):
    m = pl.program_id(0)
    nm = pl.num_programs(0)
    z = z_ref[...]
    zq = zq_ref[...]
    st_out[...] = z + (zq - z)
    diff = z - zq
    s = jnp.sum(diff * diff, keepdims=True)[:1, :1]

    @pl.when(m == 0)
    def _():
        ss_s[...] = s
        c = jnp.sum(cnt_ref[...], axis=0, keepdims=True)
        p = c * (1.0 / 8192.0)
        ent = jnp.sum(p * jnp.log(p + EPS_LOG), keepdims=True)[:1, :1]
        perp_out[...] = jnp.exp(-ent)

    @pl.when(m > 0)
    def _():
        ss_s[...] = ss_s[...] + s

    @pl.when(m == nm - 1)
    def _():
        mean = ss_s[...] * (1.0 / (8192.0 * CODE_DIM))
        loss_out[...] = mean * COMMITMENT_COST + mean


def _epilogue_call(z_flat, zq_flat, counts):
    M = z_flat.shape[0]
    grid = (M // M_BLK,)
    return pl.pallas_call(
        _epi_body,
        grid=grid,
        in_specs=[
            pl.BlockSpec((M_BLK, CODE_DIM), lambda m: (m, 0)),
            pl.BlockSpec((M_BLK, CODE_DIM), lambda m: (m, 0)),
            pl.BlockSpec((2, N_CODES), lambda m: (0, 0)),
        ],
        out_specs=[
            pl.BlockSpec((M_BLK, CODE_DIM), lambda m: (m, 0)),
            pl.BlockSpec((1, 1), lambda m: (0, 0)),
            pl.BlockSpec((1, 1), lambda m: (0, 0)),
        ],
        out_shape=[
            jax.ShapeDtypeStruct((M, CODE_DIM), jnp.float32),
            jax.ShapeDtypeStruct((1, 1), jnp.float32),
            jax.ShapeDtypeStruct((1, 1), jnp.float32),
        ],
        scratch_shapes=[pltpu.VMEM((1, 1), jnp.float32)],
    )(z_flat, zq_flat, counts)


def kernel(z, W):
    B, N, D = z.shape
    z_flat = z.reshape(-1, D)
    zsq = jnp.sum(z_flat ** 2, axis=1, keepdims=True)
    wsq = jnp.sum(W ** 2, axis=1)[None, :]
    z2b = (2.0 * z_flat).astype(jnp.bfloat16)
    wb = W.astype(jnp.bfloat16)

    idx2 = _argmin_call(z2b, wb, zsq, wsq)
    indices_flat = idx2[:, 0]
    indices = indices_flat.reshape(B, N)

    idx2d = indices_flat.reshape(8192 // 128, 128)
    zeros = jnp.zeros((N_CODES,), jnp.float32)
    zq_flat, counts = _sc_gather_counts(W, idx2d, zeros)

    zq_st_flat, loss, perp = _epilogue_call(z_flat, zq_flat, counts)
    z_q_st = zq_st_flat.reshape(B, N, D)
    return (z_q_st, indices, loss.reshape(()), perp.reshape(()))
